# baseline (speedup 1.0000x reference)
import functools

import jax
import jax.numpy as jnp
from jax import lax
from jax.experimental import pallas as pl
from jax.experimental.pallas import tpu as pltpu
from jax.experimental.pallas import tpu_sc as plsc

ND = 128
HID = 128
NB = 2000
EB = 2000
K = 128


def _stage1_body(x_ref, w1t_ref, b1_ref, w2t_ref, b2_ref, w3at_ref, w3bt_ref,
                 a_ref, b_ref, hxsum_ref):
    i = pl.program_id(0)
    x = x_ref[...]
    h = jnp.maximum(
        jnp.dot(x, w1t_ref[...], preferred_element_type=jnp.float32)
        + b1_ref[...], 0.0)
    hx = jnp.maximum(
        jnp.dot(h, w2t_ref[...], preferred_element_type=jnp.float32)
        + b2_ref[...], 0.0)
    a_ref[...] = jnp.dot(
        x, w3at_ref[...], preferred_element_type=jnp.float32
    ).astype(jnp.bfloat16)
    b_ref[...] = jnp.dot(
        x, w3bt_ref[...], preferred_element_type=jnp.float32
    ).astype(jnp.bfloat16)

    @pl.when(i == 0)
    def _():
        hxsum_ref[...] = jnp.zeros_like(hxsum_ref)

    hxsum_ref[...] += jnp.sum(hx, axis=0, keepdims=True)


def _stage1(X, W1t, b1, W2t, b2, W3at, W3bt):
    n = X.shape[0]
    grid = n // NB
    full = lambda i: (0, 0)
    return pl.pallas_call(
        _stage1_body,
        grid=(grid,),
        in_specs=[
            pl.BlockSpec((NB, ND), lambda i: (i, 0)),
            pl.BlockSpec((ND, HID), full),
            pl.BlockSpec((1, HID), full),
            pl.BlockSpec((HID, HID), full),
            pl.BlockSpec((1, HID), full),
            pl.BlockSpec((ND, HID), full),
            pl.BlockSpec((ND, HID), full),
        ],
        out_specs=[
            pl.BlockSpec((NB, HID), lambda i: (i, 0)),
            pl.BlockSpec((NB, HID), lambda i: (i, 0)),
            pl.BlockSpec((1, HID), full),
        ],
        out_shape=[
            jax.ShapeDtypeStruct((n, HID), jnp.bfloat16),
            jax.ShapeDtypeStruct((n, HID), jnp.bfloat16),
            jax.ShapeDtypeStruct((1, HID), jnp.float32),
        ],
    )(X, W1t, b1, W2t, b2, W3at, W3bt)


NBUF = 4
PW = HID // 2


def _stage2(A, B, src2d, dst2d):
    info = plsc.get_sparse_core_info()
    nc, ns = info.num_cores, info.num_subcores
    nw = nc * ns
    ncht = src2d.shape[0]
    nep = ncht * K
    nch = ncht // nw
    epw = nch * K

    mesh = plsc.VectorSubcoreMesh(core_axis_name="c", subcore_axis_name="s")

    @functools.partial(
        pl.kernel,
        mesh=mesh,
        out_type=(jax.ShapeDtypeStruct((nep, PW), jnp.int32),
                  jax.ShapeDtypeStruct((nep, PW), jnp.int32)),
        scratch_types=(
            [pltpu.VMEM((nch, K), jnp.int32)] * 2
            + [pltpu.VMEM((K, PW), jnp.int32)] * (2 * NBUF)
            + [pltpu.SemaphoreType.DMA] * (4 * NBUF)
        ),
        compiler_params=pltpu.CompilerParams(use_tc_tiling_on_sc=False),
    )
    def gather_kernel(a_hbm, b_hbm, src_hbm, dst_hbm, ga_hbm, gb_hbm,
                      si_v, di_v, *bufs_and_sems):
        ra = bufs_and_sems[0:NBUF]
        rb = bufs_and_sems[NBUF:2 * NBUF]
        sa = bufs_and_sems[2 * NBUF:3 * NBUF]
        sb = bufs_and_sems[3 * NBUF:4 * NBUF]
        wa = bufs_and_sems[4 * NBUF:5 * NBUF]
        wb = bufs_and_sems[5 * NBUF:6 * NBUF]

        wid = lax.axis_index("s") * nc + lax.axis_index("c")
        base = wid * epw
        crow = wid * nch

        pltpu.sync_copy(src_hbm.at[pl.ds(crow, nch)], si_v)
        pltpu.sync_copy(dst_hbm.at[pl.ds(crow, nch)], di_v)

        def start_gather(j, b):
            pltpu.make_async_copy(a_hbm.at[si_v.at[j]], ra[b], sa[b]).start()
            pltpu.make_async_copy(b_hbm.at[di_v.at[j]], rb[b], sb[b]).start()

        def wait_gather(j, b):
            pltpu.make_async_copy(a_hbm.at[si_v.at[j]], ra[b], sa[b]).wait()
            pltpu.make_async_copy(b_hbm.at[di_v.at[j]], rb[b], sb[b]).wait()

        def start_wb(j, b):
            off = base + j * K
            pltpu.make_async_copy(ra[b], ga_hbm.at[pl.ds(off, K)], wa[b]).start()
            pltpu.make_async_copy(rb[b], gb_hbm.at[pl.ds(off, K)], wb[b]).start()

        def wait_wb(j, b):
            off = base + j * K
            pltpu.make_async_copy(ra[b], ga_hbm.at[pl.ds(off, K)], wa[b]).wait()
            pltpu.make_async_copy(rb[b], gb_hbm.at[pl.ds(off, K)], wb[b]).wait()

        for p in range(NBUF - 1):
            start_gather(p, p)

        @pl.loop(0, nch, step=NBUF)
        def _(j0):
            for b in range(NBUF):
                j = j0 + b
                wait_gather(j, b)
                start_wb(j, b)
                nb = (b + NBUF - 1) % NBUF

                @pl.when(j > 0)
                def _():
                    wait_wb(j - 1, nb)

                q = j + NBUF - 1

                @pl.when(q < nch)
                def _():
                    start_gather(q, nb)

        wait_wb(nch - 1, (nch - 1) % NBUF)

    return gather_kernel(A, B, src2d, dst2d)


def _stage3_body(ga_ref, gb_ref, e_ref, w3ct_ref, b3_ref, w4t_ref, b4_ref,
                 wpt_ref, bp_ref, gs_ref, msum_ref, ctx_ref):
    i = pl.program_id(0)
    e = e_ref[...]
    gate = jnp.clip(1.0 + gs_ref[0, 0] * e[:, 2:3], 0.0, 3.0)
    h1 = jnp.maximum(
        ga_ref[...].astype(jnp.float32) + gb_ref[...].astype(jnp.float32)
        + jnp.dot(e, w3ct_ref[...], preferred_element_type=jnp.float32)
        + b3_ref[...], 0.0)
    m = jnp.maximum(
        jnp.dot(h1, w4t_ref[...], preferred_element_type=jnp.float32)
        + b4_ref[...], 0.0) * gate
    p = (jnp.dot(e, wpt_ref[...], preferred_element_type=jnp.float32)
         + bp_ref[...]) * gate

    @pl.when(i == 0)
    def _():
        msum_ref[...] = jnp.zeros_like(msum_ref)
        ctx_ref[...] = jnp.zeros_like(ctx_ref)

    msum_ref[...] += jnp.sum(m, axis=0, keepdims=True)
    ctx_ref[...] += jnp.sum(p, axis=0, keepdims=True)


def _stage3(GA, GB, E, W3ct, b3, W4t, b4, Wpt, bp, gs):
    ne = E.shape[0]
    grid = ne // EB
    full = lambda i: (0, 0)
    return pl.pallas_call(
        _stage3_body,
        grid=(grid,),
        in_specs=[
            pl.BlockSpec((EB, HID), lambda i: (i, 0)),
            pl.BlockSpec((EB, HID), lambda i: (i, 0)),
            pl.BlockSpec((EB, 4), lambda i: (i, 0)),
            pl.BlockSpec((4, HID), full),
            pl.BlockSpec((1, HID), full),
            pl.BlockSpec((HID, HID), full),
            pl.BlockSpec((1, HID), full),
            pl.BlockSpec((4, HID), full),
            pl.BlockSpec((1, HID), full),
            pl.BlockSpec((1, 1), full),
        ],
        out_specs=[
            pl.BlockSpec((1, HID), full),
            pl.BlockSpec((1, HID), full),
        ],
        out_shape=[
            jax.ShapeDtypeStruct((1, HID), jnp.float32),
            jax.ShapeDtypeStruct((1, HID), jnp.float32),
        ],
    )(GA, GB, E, W3ct, b3, W4t, b4, Wpt, bp, gs)


def _stage4_body(hxsum_ref, msum_ref, ctxsum_ref, wr1at_ref, wr1bt_ref,
                 br1_ref, wr2t_ref, br2_ref, out_ref, *, inv_n, inv_ne):
    hmean = (hxsum_ref[...] + 2.0 * msum_ref[...]) * inv_n
    ctx = ctxsum_ref[...] * inv_ne
    h = jnp.maximum(
        jnp.dot(hmean, wr1at_ref[...], preferred_element_type=jnp.float32)
        + jnp.dot(ctx, wr1bt_ref[...], preferred_element_type=jnp.float32)
        + br1_ref[...], 0.0)
    out_ref[...] = (jnp.dot(h, wr2t_ref[...],
                            preferred_element_type=jnp.float32) + br2_ref[...])


def _stage4(hxsum, msum, ctxsum, Wr1at, Wr1bt, br1, Wr2t, br2, n, ne):
    body = functools.partial(_stage4_body, inv_n=1.0 / n,
                             inv_ne=1.0 / (ne + 1e-06))
    return pl.pallas_call(
        body,
        out_shape=jax.ShapeDtypeStruct((1, 2), jnp.float32),
    )(hxsum, msum, ctxsum, Wr1at, Wr1bt, br1, Wr2t, br2)


def kernel(X, edges, E, W1, b1, W2, b2, W3, b3, W4, b4, Wp, bp,
           Wr1, br1, Wr2, br2, gate_scale):
    n = X.shape[0]
    ne = edges.shape[0]

    W1t = W1.T
    W2t = W2.T
    W3at = W3[:, :ND].T
    W3bt = W3[:, ND:2 * ND].T
    W3ct = W3[:, 2 * ND:].T
    W4t = W4.T
    Wpt = Wp.T
    Wr1at = Wr1[:, :HID].T
    Wr1bt = Wr1[:, HID:].T
    Wr2t = Wr2.T
    b1r = b1[None, :]
    b2r = b2[None, :]
    b3r = b3[None, :]
    b4r = b4[None, :]
    bpr = bp[None, :]
    br1r = br1[None, :]
    br2r = br2[None, :]
    gs = jnp.reshape(gate_scale.astype(jnp.float32), (1, 1))

    unit = 32 * K
    nep = ((ne + unit - 1) // unit) * unit
    src = edges[:, 0]
    dst = edges[:, 1]
    pad = jnp.zeros((nep - ne,), jnp.int32)
    src2d = jnp.concatenate([src, pad]).reshape(nep // K, K)
    dst2d = jnp.concatenate([dst, pad]).reshape(nep // K, K)

    A, B, hxsum = _stage1(X, W1t, b1r, W2t, b2r, W3at, W3bt)
    Ap = jax.lax.bitcast_convert_type(A.reshape(n, PW, 2), jnp.int32)
    Bp = jax.lax.bitcast_convert_type(B.reshape(n, PW, 2), jnp.int32)
    GAp, GBp = _stage2(Ap, Bp, src2d, dst2d)
    GA = jax.lax.bitcast_convert_type(GAp, jnp.bfloat16).reshape(nep, HID)
    GB = jax.lax.bitcast_convert_type(GBp, jnp.bfloat16).reshape(nep, HID)
    msum, ctxsum = _stage3(GA, GB, E, W3ct, b3r, W4t, b4r, Wpt, bpr, gs)
    return _stage4(hxsum, msum, ctxsum, Wr1at, Wr1bt, br1r, Wr2t, br2r, n, ne)


# speedup vs baseline: 2.2383x; 2.2383x over previous
import functools

import jax
import jax.numpy as jnp
from jax import lax
from jax.experimental import pallas as pl
from jax.experimental.pallas import tpu as pltpu
from jax.experimental.pallas import tpu_sc as plsc

ND = 128
HID = 128
NB = 2000
EB = 2000
K = 64


def _stage1_body(x_ref, w1t_ref, b1_ref, w2t_ref, b2_ref, w3at_ref, w3bt_ref,
                 a_ref, b_ref, hxsum_ref):
    i = pl.program_id(0)
    x = x_ref[...]
    h = jnp.maximum(
        jnp.dot(x, w1t_ref[...], preferred_element_type=jnp.float32)
        + b1_ref[...], 0.0)
    hx = jnp.maximum(
        jnp.dot(h, w2t_ref[...], preferred_element_type=jnp.float32)
        + b2_ref[...], 0.0)
    a_ref[...] = jnp.dot(x, w3at_ref[...], preferred_element_type=jnp.float32)
    b_ref[...] = jnp.dot(x, w3bt_ref[...], preferred_element_type=jnp.float32)

    @pl.when(i == 0)
    def _():
        hxsum_ref[...] = jnp.zeros_like(hxsum_ref)

    hxsum_ref[...] += jnp.sum(hx, axis=0, keepdims=True)


def _stage1(X, W1t, b1, W2t, b2, W3at, W3bt):
    n = X.shape[0]
    grid = n // NB
    full = lambda i: (0, 0)
    return pl.pallas_call(
        _stage1_body,
        grid=(grid,),
        in_specs=[
            pl.BlockSpec((NB, ND), lambda i: (i, 0)),
            pl.BlockSpec((ND, HID), full),
            pl.BlockSpec((1, HID), full),
            pl.BlockSpec((HID, HID), full),
            pl.BlockSpec((1, HID), full),
            pl.BlockSpec((ND, HID), full),
            pl.BlockSpec((ND, HID), full),
        ],
        out_specs=[
            pl.BlockSpec((NB, HID), lambda i: (i, 0)),
            pl.BlockSpec((NB, HID), lambda i: (i, 0)),
            pl.BlockSpec((1, HID), full),
        ],
        out_shape=[
            jax.ShapeDtypeStruct((n, HID), jnp.float32),
            jax.ShapeDtypeStruct((n, HID), jnp.float32),
            jax.ShapeDtypeStruct((1, HID), jnp.float32),
        ],
    )(X, W1t, b1, W2t, b2, W3at, W3bt)


NBUF = 4
PW = HID


def _stage2(A, B, srcp, dstp):
    info = plsc.get_sparse_core_info()
    nc, ns = info.num_cores, info.num_subcores
    nw = nc * ns
    nep = srcp.shape[0]
    epw = nep // nw
    nch = epw // K

    mesh = plsc.VectorSubcoreMesh(core_axis_name="c", subcore_axis_name="s")

    @functools.partial(
        pl.kernel,
        mesh=mesh,
        out_type=(jax.ShapeDtypeStruct((nep, PW), jnp.float32),
                  jax.ShapeDtypeStruct((nep, PW), jnp.float32)),
        scratch_types=(
            [pltpu.VMEM((epw,), jnp.int32)] * 2
            + [pltpu.VMEM((K, PW), jnp.float32)] * (2 * NBUF)
            + [pltpu.SemaphoreType.DMA] * (4 * NBUF)
        ),
    )
    def gather_kernel(a_hbm, b_hbm, src_hbm, dst_hbm, ga_hbm, gb_hbm,
                      si_v, di_v, *bufs_and_sems):
        ra = bufs_and_sems[0:NBUF]
        rb = bufs_and_sems[NBUF:2 * NBUF]
        sa = bufs_and_sems[2 * NBUF:3 * NBUF]
        sb = bufs_and_sems[3 * NBUF:4 * NBUF]
        wa = bufs_and_sems[4 * NBUF:5 * NBUF]
        wb = bufs_and_sems[5 * NBUF:6 * NBUF]

        wid = lax.axis_index("s") * nc + lax.axis_index("c")
        base = pl.multiple_of(wid * epw, K * 8)

        pltpu.sync_copy(src_hbm.at[pl.ds(base, epw)], si_v)
        pltpu.sync_copy(dst_hbm.at[pl.ds(base, epw)], di_v)

        def _idx(slab, j):
            return slab.at[pl.ds(pl.multiple_of(j * K, 8), K)]

        def start_gather(j, b):
            pltpu.make_async_copy(a_hbm.at[_idx(si_v, j)], ra[b], sa[b]).start()
            pltpu.make_async_copy(b_hbm.at[_idx(di_v, j)], rb[b], sb[b]).start()

        def wait_gather(j, b):
            pltpu.make_async_copy(a_hbm.at[_idx(si_v, j)], ra[b], sa[b]).wait()
            pltpu.make_async_copy(b_hbm.at[_idx(di_v, j)], rb[b], sb[b]).wait()

        def start_wb(j, b):
            off = pl.multiple_of(base + j * K, 8)
            pltpu.make_async_copy(ra[b], ga_hbm.at[pl.ds(off, K)], wa[b]).start()
            pltpu.make_async_copy(rb[b], gb_hbm.at[pl.ds(off, K)], wb[b]).start()

        def wait_wb(j, b):
            off = pl.multiple_of(base + j * K, 8)
            pltpu.make_async_copy(ra[b], ga_hbm.at[pl.ds(off, K)], wa[b]).wait()
            pltpu.make_async_copy(rb[b], gb_hbm.at[pl.ds(off, K)], wb[b]).wait()

        for p in range(NBUF - 1):
            start_gather(p, p)

        @pl.loop(0, nch, step=NBUF)
        def _(j0):
            for b in range(NBUF):
                j = j0 + b
                wait_gather(j, b)
                start_wb(j, b)
                nb = (b + NBUF - 1) % NBUF

                @pl.when(j > 0)
                def _():
                    wait_wb(j - 1, nb)

                q = j + NBUF - 1

                @pl.when(q < nch)
                def _():
                    start_gather(q, nb)

        wait_wb(nch - 1, (nch - 1) % NBUF)

    return gather_kernel(A, B, srcp, dstp)


def _stage3_body(ga_ref, gb_ref, e_ref, w3ct_ref, b3_ref, w4t_ref, b4_ref,
                 wpt_ref, bp_ref, gs_ref, msum_ref, ctx_ref):
    i = pl.program_id(0)
    e = e_ref[...]
    gate = jnp.clip(1.0 + gs_ref[0, 0] * e[:, 2:3], 0.0, 3.0)
    h1 = jnp.maximum(
        ga_ref[...].astype(jnp.float32) + gb_ref[...].astype(jnp.float32)
        + jnp.dot(e, w3ct_ref[...], preferred_element_type=jnp.float32)
        + b3_ref[...], 0.0)
    m = jnp.maximum(
        jnp.dot(h1, w4t_ref[...], preferred_element_type=jnp.float32)
        + b4_ref[...], 0.0) * gate
    p = (jnp.dot(e, wpt_ref[...], preferred_element_type=jnp.float32)
         + bp_ref[...]) * gate

    @pl.when(i == 0)
    def _():
        msum_ref[...] = jnp.zeros_like(msum_ref)
        ctx_ref[...] = jnp.zeros_like(ctx_ref)

    msum_ref[...] += jnp.sum(m, axis=0, keepdims=True)
    ctx_ref[...] += jnp.sum(p, axis=0, keepdims=True)


def _stage3(GA, GB, E, W3ct, b3, W4t, b4, Wpt, bp, gs):
    ne = E.shape[0]
    grid = ne // EB
    full = lambda i: (0, 0)
    return pl.pallas_call(
        _stage3_body,
        grid=(grid,),
        in_specs=[
            pl.BlockSpec((EB, HID), lambda i: (i, 0)),
            pl.BlockSpec((EB, HID), lambda i: (i, 0)),
            pl.BlockSpec((EB, 4), lambda i: (i, 0)),
            pl.BlockSpec((4, HID), full),
            pl.BlockSpec((1, HID), full),
            pl.BlockSpec((HID, HID), full),
            pl.BlockSpec((1, HID), full),
            pl.BlockSpec((4, HID), full),
            pl.BlockSpec((1, HID), full),
            pl.BlockSpec((1, 1), full),
        ],
        out_specs=[
            pl.BlockSpec((1, HID), full),
            pl.BlockSpec((1, HID), full),
        ],
        out_shape=[
            jax.ShapeDtypeStruct((1, HID), jnp.float32),
            jax.ShapeDtypeStruct((1, HID), jnp.float32),
        ],
    )(GA, GB, E, W3ct, b3, W4t, b4, Wpt, bp, gs)


def _stage4_body(hxsum_ref, msum_ref, ctxsum_ref, wr1at_ref, wr1bt_ref,
                 br1_ref, wr2t_ref, br2_ref, out_ref, *, inv_n, inv_ne):
    hmean = (hxsum_ref[...] + 2.0 * msum_ref[...]) * inv_n
    ctx = ctxsum_ref[...] * inv_ne
    h = jnp.maximum(
        jnp.dot(hmean, wr1at_ref[...], preferred_element_type=jnp.float32)
        + jnp.dot(ctx, wr1bt_ref[...], preferred_element_type=jnp.float32)
        + br1_ref[...], 0.0)
    out_ref[...] = (jnp.dot(h, wr2t_ref[...],
                            preferred_element_type=jnp.float32) + br2_ref[...])


def _stage4(hxsum, msum, ctxsum, Wr1at, Wr1bt, br1, Wr2t, br2, n, ne):
    body = functools.partial(_stage4_body, inv_n=1.0 / n,
                             inv_ne=1.0 / (ne + 1e-06))
    return pl.pallas_call(
        body,
        out_shape=jax.ShapeDtypeStruct((1, 2), jnp.float32),
    )(hxsum, msum, ctxsum, Wr1at, Wr1bt, br1, Wr2t, br2)


def kernel(X, edges, E, W1, b1, W2, b2, W3, b3, W4, b4, Wp, bp,
           Wr1, br1, Wr2, br2, gate_scale):
    n = X.shape[0]
    ne = edges.shape[0]

    W1t = W1.T
    W2t = W2.T
    W3at = W3[:, :ND].T
    W3bt = W3[:, ND:2 * ND].T
    W3ct = W3[:, 2 * ND:].T
    W4t = W4.T
    Wpt = Wp.T
    Wr1at = Wr1[:, :HID].T
    Wr1bt = Wr1[:, HID:].T
    Wr2t = Wr2.T
    b1r = b1[None, :]
    b2r = b2[None, :]
    b3r = b3[None, :]
    b4r = b4[None, :]
    bpr = bp[None, :]
    br1r = br1[None, :]
    br2r = br2[None, :]
    gs = jnp.reshape(gate_scale.astype(jnp.float32), (1, 1))

    unit = 32 * K * NBUF
    nep = ((ne + unit - 1) // unit) * unit
    src = edges[:, 0]
    dst = edges[:, 1]
    pad = jnp.zeros((nep - ne,), jnp.int32)
    srcp = jnp.concatenate([src, pad])
    dstp = jnp.concatenate([dst, pad])

    A, B, hxsum = _stage1(X, W1t, b1r, W2t, b2r, W3at, W3bt)
    GA, GB = _stage2(A, B, srcp, dstp)
    msum, ctxsum = _stage3(GA, GB, E, W3ct, b3r, W4t, b4r, Wpt, bpr, gs)
    return _stage4(hxsum, msum, ctxsum, Wr1at, Wr1bt, br1r, Wr2t, br2r, n, ne)


# speedup vs baseline: 3.9435x; 1.7618x over previous
import functools

import jax
import jax.numpy as jnp
from jax import lax
from jax.experimental import pallas as pl
from jax.experimental.pallas import tpu as pltpu
from jax.experimental.pallas import tpu_sc as plsc

ND = 128
HID = 128
NB = 2000
EB = 2000
K = 64


def _stage1_body(x_ref, w1t_ref, b1_ref, w2t_ref, b2_ref, w3at_ref, w3bt_ref,
                 a_ref, b_ref, hxsum_ref):
    i = pl.program_id(0)
    x = x_ref[...]
    h = jnp.maximum(
        jnp.dot(x, w1t_ref[...], preferred_element_type=jnp.float32)
        + b1_ref[...], 0.0)
    hx = jnp.maximum(
        jnp.dot(h, w2t_ref[...], preferred_element_type=jnp.float32)
        + b2_ref[...], 0.0)
    a_ref[...] = jnp.dot(x, w3at_ref[...], preferred_element_type=jnp.float32)
    b_ref[...] = jnp.dot(x, w3bt_ref[...], preferred_element_type=jnp.float32)

    @pl.when(i == 0)
    def _():
        hxsum_ref[...] = jnp.zeros_like(hxsum_ref)

    hxsum_ref[...] += jnp.sum(hx, axis=0, keepdims=True)


def _stage1(X, W1t, b1, W2t, b2, W3at, W3bt):
    n = X.shape[0]
    grid = n // NB
    full = lambda i: (0, 0)
    return pl.pallas_call(
        _stage1_body,
        grid=(grid,),
        in_specs=[
            pl.BlockSpec((NB, ND), lambda i: (i, 0)),
            pl.BlockSpec((ND, HID), full),
            pl.BlockSpec((1, HID), full),
            pl.BlockSpec((HID, HID), full),
            pl.BlockSpec((1, HID), full),
            pl.BlockSpec((ND, HID), full),
            pl.BlockSpec((ND, HID), full),
        ],
        out_specs=[
            pl.BlockSpec((NB, HID), lambda i: (i, 0)),
            pl.BlockSpec((NB, HID), lambda i: (i, 0)),
            pl.BlockSpec((1, HID), full),
        ],
        out_shape=[
            jax.ShapeDtypeStruct((n, HID), jnp.float32),
            jax.ShapeDtypeStruct((n, HID), jnp.float32),
            jax.ShapeDtypeStruct((1, HID), jnp.float32),
        ],
    )(X, W1t, b1, W2t, b2, W3at, W3bt)


NBUF = 4
PW = HID


def _stage2(A, B, srcp, dstp):
    info = plsc.get_sparse_core_info()
    nc, ns = info.num_cores, info.num_subcores
    nw = nc * ns
    nep = srcp.shape[0]
    epw = nep // nw
    nch = epw // K

    mesh = plsc.VectorSubcoreMesh(core_axis_name="c", subcore_axis_name="s")

    @functools.partial(
        pl.kernel,
        mesh=mesh,
        out_type=(jax.ShapeDtypeStruct((nep, PW), jnp.float32),
                  jax.ShapeDtypeStruct((nep, PW), jnp.float32)),
        scratch_types=(
            [pltpu.VMEM((epw,), jnp.int32)] * 2
            + [pltpu.VMEM((K, PW), jnp.float32)] * (2 * NBUF)
            + [pltpu.SemaphoreType.DMA] * (4 * NBUF)
        ),
    )
    def gather_kernel(a_hbm, b_hbm, src_hbm, dst_hbm, ga_hbm, gb_hbm,
                      si_v, di_v, *bufs_and_sems):
        ra = bufs_and_sems[0:NBUF]
        rb = bufs_and_sems[NBUF:2 * NBUF]
        sa = bufs_and_sems[2 * NBUF:3 * NBUF]
        sb = bufs_and_sems[3 * NBUF:4 * NBUF]
        wa = bufs_and_sems[4 * NBUF:5 * NBUF]
        wb = bufs_and_sems[5 * NBUF:6 * NBUF]

        wid = lax.axis_index("s") * nc + lax.axis_index("c")
        base = pl.multiple_of(wid * epw, K * 8)

        pltpu.sync_copy(src_hbm.at[pl.ds(base, epw)], si_v)
        pltpu.sync_copy(dst_hbm.at[pl.ds(base, epw)], di_v)

        def _idx(slab, j):
            return slab.at[pl.ds(pl.multiple_of(j * K, 8), K)]

        def start_gather(j, b):
            pltpu.make_async_copy(a_hbm.at[_idx(si_v, j)], ra[b], sa[b]).start()
            pltpu.make_async_copy(b_hbm.at[_idx(di_v, j)], rb[b], sb[b]).start()

        def wait_gather(j, b):
            pltpu.make_async_copy(a_hbm.at[_idx(si_v, j)], ra[b], sa[b]).wait()
            pltpu.make_async_copy(b_hbm.at[_idx(di_v, j)], rb[b], sb[b]).wait()

        def start_wb(j, b):
            off = pl.multiple_of(base + j * K, 8)
            pltpu.make_async_copy(ra[b], ga_hbm.at[pl.ds(off, K)], wa[b]).start()
            pltpu.make_async_copy(rb[b], gb_hbm.at[pl.ds(off, K)], wb[b]).start()

        def wait_wb(j, b):
            off = pl.multiple_of(base + j * K, 8)
            pltpu.make_async_copy(ra[b], ga_hbm.at[pl.ds(off, K)], wa[b]).wait()
            pltpu.make_async_copy(rb[b], gb_hbm.at[pl.ds(off, K)], wb[b]).wait()

        for p in range(NBUF - 1):
            start_gather(p, p)

        @pl.loop(0, nch, step=NBUF)
        def _(j0):
            for b in range(NBUF):
                j = j0 + b
                wait_gather(j, b)
                start_wb(j, b)
                nb = (b + NBUF - 1) % NBUF

                @pl.when(j > 0)
                def _():
                    wait_wb(j - 1, nb)

                q = j + NBUF - 1

                @pl.when(q < nch)
                def _():
                    start_gather(q, nb)

        wait_wb(nch - 1, (nch - 1) % NBUF)

    return gather_kernel(A, B, srcp, dstp)


def _stage3_body(ga_ref, gb_ref, e_ref, w3ct_ref, b3_ref, w4t_ref, b4_ref,
                 wpt_ref, bp_ref, gs_ref, msum_ref, ctx_ref):
    i = pl.program_id(0)
    e = e_ref[...]
    gate = jnp.clip(1.0 + gs_ref[0, 0] * e[:, 2:3], 0.0, 3.0)
    h1 = jnp.maximum(
        ga_ref[...].astype(jnp.float32) + gb_ref[...].astype(jnp.float32)
        + jnp.dot(e, w3ct_ref[...], preferred_element_type=jnp.float32)
        + b3_ref[...], 0.0)
    m = jnp.maximum(
        jnp.dot(h1, w4t_ref[...], preferred_element_type=jnp.float32)
        + b4_ref[...], 0.0) * gate
    p = (jnp.dot(e, wpt_ref[...], preferred_element_type=jnp.float32)
         + bp_ref[...]) * gate

    @pl.when(i == 0)
    def _():
        msum_ref[...] = jnp.zeros_like(msum_ref)
        ctx_ref[...] = jnp.zeros_like(ctx_ref)

    msum_ref[...] += jnp.sum(m, axis=0, keepdims=True)
    ctx_ref[...] += jnp.sum(p, axis=0, keepdims=True)


def _stage3(GA, GB, E, W3ct, b3, W4t, b4, Wpt, bp, gs):
    ne = E.shape[0]
    grid = ne // EB
    full = lambda i: (0, 0)
    return pl.pallas_call(
        _stage3_body,
        grid=(grid,),
        in_specs=[
            pl.BlockSpec((EB, HID), lambda i: (i, 0)),
            pl.BlockSpec((EB, HID), lambda i: (i, 0)),
            pl.BlockSpec((EB, 4), lambda i: (i, 0)),
            pl.BlockSpec((4, HID), full),
            pl.BlockSpec((1, HID), full),
            pl.BlockSpec((HID, HID), full),
            pl.BlockSpec((1, HID), full),
            pl.BlockSpec((4, HID), full),
            pl.BlockSpec((1, HID), full),
            pl.BlockSpec((1, 1), full),
        ],
        out_specs=[
            pl.BlockSpec((1, HID), full),
            pl.BlockSpec((1, HID), full),
        ],
        out_shape=[
            jax.ShapeDtypeStruct((1, HID), jnp.float32),
            jax.ShapeDtypeStruct((1, HID), jnp.float32),
        ],
    )(GA, GB, E, W3ct, b3, W4t, b4, Wpt, bp, gs)


def _stage4_body(hxsum_ref, msum_ref, ctxsum_ref, wr1at_ref, wr1bt_ref,
                 br1_ref, wr2t_ref, br2_ref, out_ref, *, inv_n, inv_ne):
    hmean = (hxsum_ref[...] + 2.0 * msum_ref[...]) * inv_n
    ctx = ctxsum_ref[...] * inv_ne
    h = jnp.maximum(
        jnp.dot(hmean, wr1at_ref[...], preferred_element_type=jnp.float32)
        + jnp.dot(ctx, wr1bt_ref[...], preferred_element_type=jnp.float32)
        + br1_ref[...], 0.0)
    out_ref[...] = (jnp.dot(h, wr2t_ref[...],
                            preferred_element_type=jnp.float32) + br2_ref[...])


def _stage4(hxsum, msum, ctxsum, Wr1at, Wr1bt, br1, Wr2t, br2, n, ne):
    body = functools.partial(_stage4_body, inv_n=1.0 / n,
                             inv_ne=1.0 / (ne + 1e-06))
    return pl.pallas_call(
        body,
        out_shape=jax.ShapeDtypeStruct((1, 2), jnp.float32),
    )(hxsum, msum, ctxsum, Wr1at, Wr1bt, br1, Wr2t, br2)


def kernel(X, edges, E, W1, b1, W2, b2, W3, b3, W4, b4, Wp, bp,
           Wr1, br1, Wr2, br2, gate_scale):
    n = X.shape[0]
    ne = edges.shape[0]

    W1t = W1.T
    W2t = W2.T
    W3at = W3[:, :ND].T
    W3bt = W3[:, ND:2 * ND].T
    W3ct = W3[:, 2 * ND:].T
    W4t = W4.T
    Wpt = Wp.T
    Wr1at = Wr1[:, :HID].T
    Wr1bt = Wr1[:, HID:].T
    Wr2t = Wr2.T
    b1r = b1[None, :]
    b2r = b2[None, :]
    b3r = b3[None, :]
    b4r = b4[None, :]
    bpr = bp[None, :]
    br1r = br1[None, :]
    br2r = br2[None, :]
    gs = jnp.reshape(gate_scale.astype(jnp.float32), (1, 1))

    unit = 32 * K * NBUF
    nep = ((ne + unit - 1) // unit) * unit
    src = edges[:, 0]
    dst = edges[:, 1]
    pad = (jnp.arange(nep - ne, dtype=jnp.int32) * 8) % n
    srcp = jnp.concatenate([src, pad])
    dstp = jnp.concatenate([dst, pad])

    A, B, hxsum = _stage1(X, W1t, b1r, W2t, b2r, W3at, W3bt)
    GA, GB = _stage2(A, B, srcp, dstp)
    msum, ctxsum = _stage3(GA, GB, E, W3ct, b3r, W4t, b4r, Wpt, bpr, gs)
    return _stage4(hxsum, msum, ctxsum, Wr1at, Wr1bt, br1r, Wr2t, br2r, n, ne)


# speedup vs baseline: 4.7493x; 1.2043x over previous
import functools

import jax
import jax.numpy as jnp
from jax import lax
from jax.experimental import pallas as pl
from jax.experimental.pallas import tpu as pltpu
from jax.experimental.pallas import tpu_sc as plsc

ND = 128
HID = 128
ED = 4
NB = 2000
EB = 2048
K = 64
NPH = 4


def _stage1_body(x_ref, w1t_ref, b1_ref, w2t_ref, b2_ref, w3at_ref, w3bt_ref,
                 a_ref, b_ref, hxsum_ref):
    i = pl.program_id(0)
    x = x_ref[...]
    h = jnp.maximum(
        jnp.dot(x, w1t_ref[...], preferred_element_type=jnp.float32)
        + b1_ref[...], 0.0)
    hx = jnp.maximum(
        jnp.dot(h, w2t_ref[...], preferred_element_type=jnp.float32)
        + b2_ref[...], 0.0)
    a_ref[...] = jnp.dot(x, w3at_ref[...], preferred_element_type=jnp.float32)
    b_ref[...] = jnp.dot(x, w3bt_ref[...], preferred_element_type=jnp.float32)

    @pl.when(i == 0)
    def _():
        hxsum_ref[...] = jnp.zeros_like(hxsum_ref)

    hxsum_ref[...] += jnp.sum(hx, axis=0, keepdims=True)


def _stage1(X, W1t, b1, W2t, b2, W3at, W3bt):
    n = X.shape[0]
    grid = n // NB
    full = lambda i: (0, 0)
    return pl.pallas_call(
        _stage1_body,
        grid=(grid,),
        in_specs=[
            pl.BlockSpec((NB, ND), lambda i: (i, 0)),
            pl.BlockSpec((ND, HID), full),
            pl.BlockSpec((1, HID), full),
            pl.BlockSpec((HID, HID), full),
            pl.BlockSpec((1, HID), full),
            pl.BlockSpec((ND, HID), full),
            pl.BlockSpec((ND, HID), full),
        ],
        out_specs=[
            pl.BlockSpec((NB, HID), lambda i: (i, 0)),
            pl.BlockSpec((NB, HID), lambda i: (i, 0)),
            pl.BlockSpec((1, HID), full),
        ],
        out_shape=[
            jax.ShapeDtypeStruct((n, HID), jnp.float32),
            jax.ShapeDtypeStruct((n, HID), jnp.float32),
            jax.ShapeDtypeStruct((1, HID), jnp.float32),
        ],
    )(X, W1t, b1, W2t, b2, W3at, W3bt)


NBUF = 4
PW = HID


def _stage2(A, B, srcp, dstp):
    info = plsc.get_sparse_core_info()
    nc, ns = info.num_cores, info.num_subcores
    nw = nc * ns
    nep = srcp.shape[0]
    epw = nep // nw
    nch = epw // K

    mesh = plsc.VectorSubcoreMesh(core_axis_name="c", subcore_axis_name="s")

    @functools.partial(
        pl.kernel,
        mesh=mesh,
        out_type=(jax.ShapeDtypeStruct((nep, PW), jnp.float32),
                  jax.ShapeDtypeStruct((nep, PW), jnp.float32)),
        scratch_types=(
            [pltpu.VMEM((epw,), jnp.int32)] * 2
            + [pltpu.VMEM((K, PW), jnp.float32)] * (2 * NBUF)
            + [pltpu.SemaphoreType.DMA] * (4 * NBUF)
        ),
    )
    def gather_kernel(a_hbm, b_hbm, src_hbm, dst_hbm, ga_hbm, gb_hbm,
                      si_v, di_v, *bufs_and_sems):
        ra = bufs_and_sems[0:NBUF]
        rb = bufs_and_sems[NBUF:2 * NBUF]
        sa = bufs_and_sems[2 * NBUF:3 * NBUF]
        sb = bufs_and_sems[3 * NBUF:4 * NBUF]
        wa = bufs_and_sems[4 * NBUF:5 * NBUF]
        wb = bufs_and_sems[5 * NBUF:6 * NBUF]

        wid = lax.axis_index("s") * nc + lax.axis_index("c")
        base = pl.multiple_of(wid * epw, K * 8)

        pltpu.sync_copy(src_hbm.at[pl.ds(base, epw)], si_v)
        pltpu.sync_copy(dst_hbm.at[pl.ds(base, epw)], di_v)

        def _idx(slab, j):
            return slab.at[pl.ds(pl.multiple_of(j * K, 8), K)]

        def start_gather(j, b):
            pltpu.make_async_copy(a_hbm.at[_idx(si_v, j)], ra[b], sa[b]).start()
            pltpu.make_async_copy(b_hbm.at[_idx(di_v, j)], rb[b], sb[b]).start()

        def wait_gather(j, b):
            pltpu.make_async_copy(a_hbm.at[_idx(si_v, j)], ra[b], sa[b]).wait()
            pltpu.make_async_copy(b_hbm.at[_idx(di_v, j)], rb[b], sb[b]).wait()

        def start_wb(j, b):
            off = pl.multiple_of(base + j * K, 8)
            pltpu.make_async_copy(ra[b], ga_hbm.at[pl.ds(off, K)], wa[b]).start()
            pltpu.make_async_copy(rb[b], gb_hbm.at[pl.ds(off, K)], wb[b]).start()

        def wait_wb(j, b):
            off = pl.multiple_of(base + j * K, 8)
            pltpu.make_async_copy(ra[b], ga_hbm.at[pl.ds(off, K)], wa[b]).wait()
            pltpu.make_async_copy(rb[b], gb_hbm.at[pl.ds(off, K)], wb[b]).wait()

        for p in range(NBUF - 1):
            start_gather(p, p)

        @pl.loop(0, nch, step=NBUF)
        def _(j0):
            for b in range(NBUF):
                j = j0 + b
                wait_gather(j, b)
                start_wb(j, b)
                nb = (b + NBUF - 1) % NBUF

                @pl.when(j > 0)
                def _():
                    wait_wb(j - 1, nb)

                q = j + NBUF - 1

                @pl.when(q < nch)
                def _():
                    start_gather(q, nb)

        wait_wb(nch - 1, (nch - 1) % NBUF)

    return gather_kernel(A, B, srcp, dstp)


def _stage3_body(ga_ref, gb_ref, et_ref, mask_ref, w3ct_ref, b3_ref, w4t_ref,
                 b4_ref, wpt_ref, bp_ref, gs_ref, msum_ref, ctx_ref):
    i = pl.program_id(0)
    et = et_ref[...]
    gate = jnp.clip(1.0 + gs_ref[0, 0] * et[2:3, :], 0.0, 3.0) * mask_ref[...]
    dn = (((0,), (0,)), ((), ()))
    ec = lax.dot_general(et, w3ct_ref[...], dn,
                         preferred_element_type=jnp.float32)
    h1 = jnp.maximum(ga_ref[...] + gb_ref[...] + ec + b3_ref[...], 0.0)
    mm = jnp.maximum(
        jnp.dot(h1, w4t_ref[...], preferred_element_type=jnp.float32)
        + b4_ref[...], 0.0)
    p = lax.dot_general(et, wpt_ref[...], dn,
                        preferred_element_type=jnp.float32) + bp_ref[...]

    @pl.when(i == 0)
    def _():
        msum_ref[...] = jnp.zeros_like(msum_ref)
        ctx_ref[...] = jnp.zeros_like(ctx_ref)

    msum_ref[...] += jnp.dot(gate, mm, preferred_element_type=jnp.float32)
    ctx_ref[...] += jnp.dot(gate, p, preferred_element_type=jnp.float32)


def _stage3(GA, GB, Et, mask, W3ct, b3, W4t, b4, Wpt, bp, gs):
    npe = GA.shape[0]
    grid = npe // EB
    full = lambda i: (0, 0)
    return pl.pallas_call(
        _stage3_body,
        grid=(grid,),
        in_specs=[
            pl.BlockSpec((EB, HID), lambda i: (i, 0)),
            pl.BlockSpec((EB, HID), lambda i: (i, 0)),
            pl.BlockSpec((4, EB), lambda i: (0, i)),
            pl.BlockSpec((1, EB), lambda i: (0, i)),
            pl.BlockSpec((4, HID), full),
            pl.BlockSpec((1, HID), full),
            pl.BlockSpec((HID, HID), full),
            pl.BlockSpec((1, HID), full),
            pl.BlockSpec((4, HID), full),
            pl.BlockSpec((1, HID), full),
            pl.BlockSpec((1, 1), full),
        ],
        out_specs=[
            pl.BlockSpec((1, HID), full),
            pl.BlockSpec((1, HID), full),
        ],
        out_shape=[
            jax.ShapeDtypeStruct((1, HID), jnp.float32),
            jax.ShapeDtypeStruct((1, HID), jnp.float32),
        ],
    )(GA, GB, Et, mask, W3ct, b3, W4t, b4, Wpt, bp, gs)


def _stage4_body(hxsum_ref, msum_ref, ctxsum_ref, wr1at_ref, wr1bt_ref,
                 br1_ref, wr2t_ref, br2_ref, out_ref, *, inv_n, inv_ne):
    msum = jnp.sum(msum_ref[...], axis=0, keepdims=True)
    ctxsum = jnp.sum(ctxsum_ref[...], axis=0, keepdims=True)
    hmean = (hxsum_ref[...] + 2.0 * msum) * inv_n
    ctx = ctxsum * inv_ne
    h = jnp.maximum(
        jnp.dot(hmean, wr1at_ref[...], preferred_element_type=jnp.float32)
        + jnp.dot(ctx, wr1bt_ref[...], preferred_element_type=jnp.float32)
        + br1_ref[...], 0.0)
    out_ref[...] = (jnp.dot(h, wr2t_ref[...],
                            preferred_element_type=jnp.float32) + br2_ref[...])


def _stage4(hxsum, msum, ctxsum, Wr1at, Wr1bt, br1, Wr2t, br2, n, ne):
    body = functools.partial(_stage4_body, inv_n=1.0 / n,
                             inv_ne=1.0 / (ne + 1e-06))
    return pl.pallas_call(
        body,
        out_shape=jax.ShapeDtypeStruct((1, 2), jnp.float32),
    )(hxsum, msum, ctxsum, Wr1at, Wr1bt, br1, Wr2t, br2)


def kernel(X, edges, E, W1, b1, W2, b2, W3, b3, W4, b4, Wp, bp,
           Wr1, br1, Wr2, br2, gate_scale):
    n = X.shape[0]
    ne = edges.shape[0]

    W1t = W1.T
    W2t = W2.T
    W3at = W3[:, :ND].T
    W3bt = W3[:, ND:2 * ND].T
    W3ct = W3[:, 2 * ND:].T
    W4t = W4.T
    Wpt = Wp.T
    Wr1at = Wr1[:, :HID].T
    Wr1bt = Wr1[:, HID:].T
    Wr2t = Wr2.T
    b1r = b1[None, :]
    b2r = b2[None, :]
    b3r = b3[None, :]
    b4r = b4[None, :]
    bpr = bp[None, :]
    br1r = br1[None, :]
    br2r = br2[None, :]
    gs = jnp.reshape(gate_scale.astype(jnp.float32), (1, 1))

    unit = 32 * K * NBUF * NPH
    nep = ((ne + unit - 1) // unit) * unit
    src = edges[:, 0]
    dst = edges[:, 1]
    pad = (jnp.arange(nep - ne, dtype=jnp.int32) * 8) % n
    srcp = jnp.concatenate([src, pad])
    dstp = jnp.concatenate([dst, pad])
    Et = jnp.concatenate([E.T, jnp.zeros((ED, nep - ne), jnp.float32)], axis=1)
    mask = (jnp.arange(nep, dtype=jnp.int32) < ne).astype(jnp.float32)[None, :]

    A, B, hxsum = _stage1(X, W1t, b1r, W2t, b2r, W3at, W3bt)

    ppe = nep // NPH
    msums, ctxs = [], []
    for p in range(NPH):
        sl = slice(p * ppe, (p + 1) * ppe)
        GA, GB = _stage2(A, B, srcp[sl], dstp[sl])
        ms, cs = _stage3(GA, GB, Et[:, sl], mask[:, sl], W3ct, b3r, W4t, b4r,
                         Wpt, bpr, gs)
        msums.append(ms)
        ctxs.append(cs)
    msum = jnp.concatenate(msums, axis=0)
    ctxsum = jnp.concatenate(ctxs, axis=0)
    return _stage4(hxsum, msum, ctxsum, Wr1at, Wr1bt, br1r, Wr2t, br2r, n, ne)


# speedup vs baseline: 5.8536x; 1.2325x over previous
import functools

import jax
import jax.numpy as jnp
from jax import lax
from jax.experimental import pallas as pl
from jax.experimental.pallas import tpu as pltpu
from jax.experimental.pallas import tpu_sc as plsc

ND = 128
HID = 128
ED = 4
NB = 2000
EB = 2048
K = 64
NPH = 4


def _stage1_body(x_ref, w1t_ref, b1_ref, w2t_ref, b2_ref, w3at_ref, w3bt_ref,
                 a_ref, b_ref, hxsum_ref):
    i = pl.program_id(0)
    x = x_ref[...]
    h = jnp.maximum(
        jnp.dot(x, w1t_ref[...], preferred_element_type=jnp.float32)
        + b1_ref[...], 0.0)
    hx = jnp.maximum(
        jnp.dot(h, w2t_ref[...], preferred_element_type=jnp.float32)
        + b2_ref[...], 0.0)
    a_ref[...] = jnp.dot(x, w3at_ref[...], preferred_element_type=jnp.float32)
    b_ref[...] = jnp.dot(x, w3bt_ref[...], preferred_element_type=jnp.float32)

    @pl.when(i == 0)
    def _():
        hxsum_ref[...] = jnp.zeros_like(hxsum_ref)

    hxsum_ref[...] += jnp.sum(hx, axis=0, keepdims=True)


def _stage1(X, W1t, b1, W2t, b2, W3at, W3bt):
    n = X.shape[0]
    grid = n // NB
    full = lambda i: (0, 0)
    return pl.pallas_call(
        _stage1_body,
        grid=(grid,),
        in_specs=[
            pl.BlockSpec((NB, ND), lambda i: (i, 0)),
            pl.BlockSpec((ND, HID), full),
            pl.BlockSpec((1, HID), full),
            pl.BlockSpec((HID, HID), full),
            pl.BlockSpec((1, HID), full),
            pl.BlockSpec((ND, HID), full),
            pl.BlockSpec((ND, HID), full),
        ],
        out_specs=[
            pl.BlockSpec((NB, HID), lambda i: (i, 0)),
            pl.BlockSpec((NB, HID), lambda i: (i, 0)),
            pl.BlockSpec((1, HID), full),
        ],
        out_shape=[
            jax.ShapeDtypeStruct((n, HID), jnp.float32),
            jax.ShapeDtypeStruct((n, HID), jnp.float32),
            jax.ShapeDtypeStruct((1, HID), jnp.float32),
        ],
    )(X, W1t, b1, W2t, b2, W3at, W3bt)


NBUF = 4
PW = HID


def _stage2(A, B, srcp, dstp):
    info = plsc.get_sparse_core_info()
    nc, ns = info.num_cores, info.num_subcores
    nw = nc * ns
    nep = srcp.shape[0]
    epw = nep // nw
    nch = epw // K

    mesh = plsc.VectorSubcoreMesh(core_axis_name="c", subcore_axis_name="s")

    @functools.partial(
        pl.kernel,
        mesh=mesh,
        out_type=jax.ShapeDtypeStruct((nep, PW), jnp.float32),
        scratch_types=(
            [pltpu.VMEM((epw,), jnp.int32)] * 2
            + [pltpu.VMEM((K, PW), jnp.float32)] * (2 * NBUF)
            + [pltpu.SemaphoreType.DMA] * (3 * NBUF)
        ),
    )
    def gather_kernel(a_hbm, b_hbm, src_hbm, dst_hbm, g_hbm,
                      si_v, di_v, *bufs_and_sems):
        ra = bufs_and_sems[0:NBUF]
        rb = bufs_and_sems[NBUF:2 * NBUF]
        sa = bufs_and_sems[2 * NBUF:3 * NBUF]
        sb = bufs_and_sems[3 * NBUF:4 * NBUF]
        wa = bufs_and_sems[4 * NBUF:5 * NBUF]

        wid = lax.axis_index("s") * nc + lax.axis_index("c")
        base = pl.multiple_of(wid * epw, K * 8)

        pltpu.sync_copy(src_hbm.at[pl.ds(base, epw)], si_v)
        pltpu.sync_copy(dst_hbm.at[pl.ds(base, epw)], di_v)

        def _idx(slab, j):
            return slab.at[pl.ds(pl.multiple_of(j * K, 8), K)]

        def start_gather(j, b):
            pltpu.make_async_copy(a_hbm.at[_idx(si_v, j)], ra[b], sa[b]).start()
            pltpu.make_async_copy(b_hbm.at[_idx(di_v, j)], rb[b], sb[b]).start()

        def wait_gather(j, b):
            pltpu.make_async_copy(a_hbm.at[_idx(si_v, j)], ra[b], sa[b]).wait()
            pltpu.make_async_copy(b_hbm.at[_idx(di_v, j)], rb[b], sb[b]).wait()

        def add_bufs(b):
            @pl.loop(0, K)
            def _(r):
                for c in range(PW // 16):
                    sl = pl.ds(c * 16, 16)
                    ra[b][r, sl] = ra[b][r, sl] + rb[b][r, sl]

        def start_wb(j, b):
            off = pl.multiple_of(base + j * K, 8)
            pltpu.make_async_copy(ra[b], g_hbm.at[pl.ds(off, K)], wa[b]).start()

        def wait_wb(j, b):
            off = pl.multiple_of(base + j * K, 8)
            pltpu.make_async_copy(ra[b], g_hbm.at[pl.ds(off, K)], wa[b]).wait()

        for p in range(NBUF - 1):
            start_gather(p, p)

        @pl.loop(0, nch, step=NBUF)
        def _(j0):
            for b in range(NBUF):
                j = j0 + b
                wait_gather(j, b)
                add_bufs(b)
                start_wb(j, b)
                nb = (b + NBUF - 1) % NBUF

                @pl.when(j > 0)
                def _():
                    wait_wb(j - 1, nb)

                q = j + NBUF - 1

                @pl.when(q < nch)
                def _():
                    start_gather(q, nb)

        wait_wb(nch - 1, (nch - 1) % NBUF)

    return gather_kernel(A, B, srcp, dstp)


def _stage3_body(g_ref, et_ref, mask_ref, w3ct_ref, b3_ref, w4t_ref,
                 b4_ref, wpt_ref, bp_ref, gs_ref, msum_ref, ctx_ref):
    i = pl.program_id(0)
    et = et_ref[...]
    gate = jnp.clip(1.0 + gs_ref[0, 0] * et[2:3, :], 0.0, 3.0) * mask_ref[...]
    dn = (((0,), (0,)), ((), ()))
    ec = lax.dot_general(et, w3ct_ref[...], dn,
                         preferred_element_type=jnp.float32)
    h1 = jnp.maximum(g_ref[...] + ec + b3_ref[...], 0.0)
    mm = jnp.maximum(
        jnp.dot(h1, w4t_ref[...], preferred_element_type=jnp.float32)
        + b4_ref[...], 0.0)
    p = lax.dot_general(et, wpt_ref[...], dn,
                        preferred_element_type=jnp.float32) + bp_ref[...]

    @pl.when(i == 0)
    def _():
        msum_ref[...] = jnp.zeros_like(msum_ref)
        ctx_ref[...] = jnp.zeros_like(ctx_ref)

    msum_ref[...] += jnp.dot(gate, mm, preferred_element_type=jnp.float32)
    ctx_ref[...] += jnp.dot(gate, p, preferred_element_type=jnp.float32)


def _stage3(G, Et, mask, W3ct, b3, W4t, b4, Wpt, bp, gs):
    npe = G.shape[0]
    grid = npe // EB
    full = lambda i: (0, 0)
    return pl.pallas_call(
        _stage3_body,
        grid=(grid,),
        in_specs=[
            pl.BlockSpec((EB, HID), lambda i: (i, 0)),
            pl.BlockSpec((4, EB), lambda i: (0, i)),
            pl.BlockSpec((1, EB), lambda i: (0, i)),
            pl.BlockSpec((4, HID), full),
            pl.BlockSpec((1, HID), full),
            pl.BlockSpec((HID, HID), full),
            pl.BlockSpec((1, HID), full),
            pl.BlockSpec((4, HID), full),
            pl.BlockSpec((1, HID), full),
            pl.BlockSpec((1, 1), full),
        ],
        out_specs=[
            pl.BlockSpec((1, HID), full),
            pl.BlockSpec((1, HID), full),
        ],
        out_shape=[
            jax.ShapeDtypeStruct((1, HID), jnp.float32),
            jax.ShapeDtypeStruct((1, HID), jnp.float32),
        ],
    )(G, Et, mask, W3ct, b3, W4t, b4, Wpt, bp, gs)


def _stage4_body(hxsum_ref, msum_ref, ctxsum_ref, wr1at_ref, wr1bt_ref,
                 br1_ref, wr2t_ref, br2_ref, out_ref, *, inv_n, inv_ne):
    msum = jnp.sum(msum_ref[...], axis=0, keepdims=True)
    ctxsum = jnp.sum(ctxsum_ref[...], axis=0, keepdims=True)
    hmean = (hxsum_ref[...] + 2.0 * msum) * inv_n
    ctx = ctxsum * inv_ne
    h = jnp.maximum(
        jnp.dot(hmean, wr1at_ref[...], preferred_element_type=jnp.float32)
        + jnp.dot(ctx, wr1bt_ref[...], preferred_element_type=jnp.float32)
        + br1_ref[...], 0.0)
    out_ref[...] = (jnp.dot(h, wr2t_ref[...],
                            preferred_element_type=jnp.float32) + br2_ref[...])


def _stage4(hxsum, msum, ctxsum, Wr1at, Wr1bt, br1, Wr2t, br2, n, ne):
    body = functools.partial(_stage4_body, inv_n=1.0 / n,
                             inv_ne=1.0 / (ne + 1e-06))
    return pl.pallas_call(
        body,
        out_shape=jax.ShapeDtypeStruct((1, 2), jnp.float32),
    )(hxsum, msum, ctxsum, Wr1at, Wr1bt, br1, Wr2t, br2)


def kernel(X, edges, E, W1, b1, W2, b2, W3, b3, W4, b4, Wp, bp,
           Wr1, br1, Wr2, br2, gate_scale):
    n = X.shape[0]
    ne = edges.shape[0]

    W1t = W1.T
    W2t = W2.T
    W3at = W3[:, :ND].T
    W3bt = W3[:, ND:2 * ND].T
    W3ct = W3[:, 2 * ND:].T
    W4t = W4.T
    Wpt = Wp.T
    Wr1at = Wr1[:, :HID].T
    Wr1bt = Wr1[:, HID:].T
    Wr2t = Wr2.T
    b1r = b1[None, :]
    b2r = b2[None, :]
    b3r = b3[None, :]
    b4r = b4[None, :]
    bpr = bp[None, :]
    br1r = br1[None, :]
    br2r = br2[None, :]
    gs = jnp.reshape(gate_scale.astype(jnp.float32), (1, 1))

    unit = 32 * K * NBUF * NPH
    nep = ((ne + unit - 1) // unit) * unit
    src = edges[:, 0]
    dst = edges[:, 1]
    pad = (jnp.arange(nep - ne, dtype=jnp.int32) * 8) % n
    srcp = jnp.concatenate([src, pad])
    dstp = jnp.concatenate([dst, pad])
    Et = jnp.concatenate([E.T, jnp.zeros((ED, nep - ne), jnp.float32)], axis=1)
    mask = (jnp.arange(nep, dtype=jnp.int32) < ne).astype(jnp.float32)[None, :]

    A, B, hxsum = _stage1(X, W1t, b1r, W2t, b2r, W3at, W3bt)

    ppe = nep // NPH
    msums, ctxs = [], []
    for p in range(NPH):
        sl = slice(p * ppe, (p + 1) * ppe)
        G = _stage2(A, B, srcp[sl], dstp[sl])
        ms, cs = _stage3(G, Et[:, sl], mask[:, sl], W3ct, b3r, W4t, b4r,
                         Wpt, bpr, gs)
        msums.append(ms)
        ctxs.append(cs)
    msum = jnp.concatenate(msums, axis=0)
    ctxsum = jnp.concatenate(ctxs, axis=0)
    return _stage4(hxsum, msum, ctxsum, Wr1at, Wr1bt, br1r, Wr2t, br2r, n, ne)


# speedup vs baseline: 6.1061x; 1.0432x over previous
import functools

import jax
import jax.numpy as jnp
from jax import lax
from jax.experimental import pallas as pl
from jax.experimental.pallas import tpu as pltpu
from jax.experimental.pallas import tpu_sc as plsc

ND = 128
HID = 128
ED = 4
NB = 2000
EB = 4096
K = 64
PHASES = (3, 4, 5, 8)


def _stage1_body(x_ref, w1t_ref, b1_ref, w2t_ref, b2_ref, w3at_ref, w3bt_ref,
                 a_ref, b_ref, hxsum_ref):
    i = pl.program_id(0)
    x = x_ref[...]
    h = jnp.maximum(
        jnp.dot(x, w1t_ref[...], preferred_element_type=jnp.float32)
        + b1_ref[...], 0.0)
    hx = jnp.maximum(
        jnp.dot(h, w2t_ref[...], preferred_element_type=jnp.float32)
        + b2_ref[...], 0.0)
    a_ref[...] = jnp.dot(x, w3at_ref[...], preferred_element_type=jnp.float32)
    b_ref[...] = jnp.dot(x, w3bt_ref[...], preferred_element_type=jnp.float32)

    @pl.when(i == 0)
    def _():
        hxsum_ref[...] = jnp.zeros_like(hxsum_ref)

    hxsum_ref[...] += jnp.sum(hx, axis=0, keepdims=True)


def _stage1(X, W1t, b1, W2t, b2, W3at, W3bt):
    n = X.shape[0]
    grid = n // NB
    full = lambda i: (0, 0)
    return pl.pallas_call(
        _stage1_body,
        grid=(grid,),
        in_specs=[
            pl.BlockSpec((NB, ND), lambda i: (i, 0)),
            pl.BlockSpec((ND, HID), full),
            pl.BlockSpec((1, HID), full),
            pl.BlockSpec((HID, HID), full),
            pl.BlockSpec((1, HID), full),
            pl.BlockSpec((ND, HID), full),
            pl.BlockSpec((ND, HID), full),
        ],
        out_specs=[
            pl.BlockSpec((NB, HID), lambda i: (i, 0)),
            pl.BlockSpec((NB, HID), lambda i: (i, 0)),
            pl.BlockSpec((1, HID), full),
        ],
        out_shape=[
            jax.ShapeDtypeStruct((n, HID), jnp.float32),
            jax.ShapeDtypeStruct((n, HID), jnp.float32),
            jax.ShapeDtypeStruct((1, HID), jnp.float32),
        ],
    )(X, W1t, b1, W2t, b2, W3at, W3bt)


NBUF = 4
PW = HID


def _stage2(A, B, srcp, dstp):
    info = plsc.get_sparse_core_info()
    nc, ns = info.num_cores, info.num_subcores
    nw = nc * ns
    nep = srcp.shape[0]
    epw = nep // nw
    nch = epw // K

    mesh = plsc.VectorSubcoreMesh(core_axis_name="c", subcore_axis_name="s")

    @functools.partial(
        pl.kernel,
        mesh=mesh,
        out_type=jax.ShapeDtypeStruct((nep, PW), jnp.float32),
        scratch_types=(
            [pltpu.VMEM((epw,), jnp.int32)] * 2
            + [pltpu.VMEM((K, PW), jnp.float32)] * (2 * NBUF)
            + [pltpu.SemaphoreType.DMA] * (3 * NBUF)
        ),
    )
    def gather_kernel(a_hbm, b_hbm, src_hbm, dst_hbm, g_hbm,
                      si_v, di_v, *bufs_and_sems):
        ra = bufs_and_sems[0:NBUF]
        rb = bufs_and_sems[NBUF:2 * NBUF]
        sa = bufs_and_sems[2 * NBUF:3 * NBUF]
        sb = bufs_and_sems[3 * NBUF:4 * NBUF]
        wa = bufs_and_sems[4 * NBUF:5 * NBUF]

        wid = lax.axis_index("s") * nc + lax.axis_index("c")
        base = pl.multiple_of(wid * epw, K * 8)

        pltpu.sync_copy(src_hbm.at[pl.ds(base, epw)], si_v)
        pltpu.sync_copy(dst_hbm.at[pl.ds(base, epw)], di_v)

        def _idx(slab, j):
            return slab.at[pl.ds(pl.multiple_of(j * K, 8), K)]

        def start_gather(j, b):
            pltpu.make_async_copy(a_hbm.at[_idx(si_v, j)], ra[b], sa[b]).start()
            pltpu.make_async_copy(b_hbm.at[_idx(di_v, j)], rb[b], sb[b]).start()

        def wait_gather(j, b):
            pltpu.make_async_copy(a_hbm.at[_idx(si_v, j)], ra[b], sa[b]).wait()
            pltpu.make_async_copy(b_hbm.at[_idx(di_v, j)], rb[b], sb[b]).wait()

        def add_bufs(b):
            @pl.loop(0, K)
            def _(r):
                for c in range(PW // 16):
                    sl = pl.ds(c * 16, 16)
                    ra[b][r, sl] = ra[b][r, sl] + rb[b][r, sl]

        def start_wb(j, b):
            off = pl.multiple_of(base + j * K, 8)
            pltpu.make_async_copy(ra[b], g_hbm.at[pl.ds(off, K)], wa[b]).start()

        def wait_wb(j, b):
            off = pl.multiple_of(base + j * K, 8)
            pltpu.make_async_copy(ra[b], g_hbm.at[pl.ds(off, K)], wa[b]).wait()

        for p in range(NBUF - 1):
            start_gather(p, p)

        @pl.loop(0, nch, step=NBUF)
        def _(j0):
            for b in range(NBUF):
                j = j0 + b
                wait_gather(j, b)
                add_bufs(b)
                start_wb(j, b)
                nb = (b + NBUF - 1) % NBUF

                @pl.when(j > 0)
                def _():
                    wait_wb(j - 1, nb)

                q = j + NBUF - 1

                @pl.when(q < nch)
                def _():
                    start_gather(q, nb)

        wait_wb(nch - 1, (nch - 1) % NBUF)

    return gather_kernel(A, B, srcp, dstp)


def _stage3_body(g_ref, et_ref, mask_ref, w3ct_ref, b3_ref, w4t_ref,
                 b4_ref, wpt_ref, bp_ref, gs_ref, msum_ref, ctx_ref):
    i = pl.program_id(0)
    et = et_ref[...]
    gate = jnp.clip(1.0 + gs_ref[0, 0] * et[2:3, :], 0.0, 3.0) * mask_ref[...]
    dn = (((0,), (0,)), ((), ()))
    ec = lax.dot_general(et, w3ct_ref[...], dn,
                         preferred_element_type=jnp.float32)
    h1 = jnp.maximum(g_ref[...] + ec + b3_ref[...], 0.0)
    mm = jnp.maximum(
        jnp.dot(h1, w4t_ref[...], preferred_element_type=jnp.float32)
        + b4_ref[...], 0.0)
    p = lax.dot_general(et, wpt_ref[...], dn,
                        preferred_element_type=jnp.float32) + bp_ref[...]

    @pl.when(i == 0)
    def _():
        msum_ref[...] = jnp.zeros_like(msum_ref)
        ctx_ref[...] = jnp.zeros_like(ctx_ref)

    msum_ref[...] += jnp.dot(gate, mm, preferred_element_type=jnp.float32)
    ctx_ref[...] += jnp.dot(gate, p, preferred_element_type=jnp.float32)


def _stage3(G, Et, mask, W3ct, b3, W4t, b4, Wpt, bp, gs):
    npe = G.shape[0]
    grid = npe // EB
    full = lambda i: (0, 0)
    return pl.pallas_call(
        _stage3_body,
        grid=(grid,),
        in_specs=[
            pl.BlockSpec((EB, HID), lambda i: (i, 0)),
            pl.BlockSpec((4, EB), lambda i: (0, i)),
            pl.BlockSpec((1, EB), lambda i: (0, i)),
            pl.BlockSpec((4, HID), full),
            pl.BlockSpec((1, HID), full),
            pl.BlockSpec((HID, HID), full),
            pl.BlockSpec((1, HID), full),
            pl.BlockSpec((4, HID), full),
            pl.BlockSpec((1, HID), full),
            pl.BlockSpec((1, 1), full),
        ],
        out_specs=[
            pl.BlockSpec((1, HID), full),
            pl.BlockSpec((1, HID), full),
        ],
        out_shape=[
            jax.ShapeDtypeStruct((1, HID), jnp.float32),
            jax.ShapeDtypeStruct((1, HID), jnp.float32),
        ],
    )(G, Et, mask, W3ct, b3, W4t, b4, Wpt, bp, gs)


def _stage4_body(hxsum_ref, msum_ref, ctxsum_ref, wr1at_ref, wr1bt_ref,
                 br1_ref, wr2t_ref, br2_ref, out_ref, *, inv_n, inv_ne):
    msum = jnp.sum(msum_ref[...], axis=0, keepdims=True)
    ctxsum = jnp.sum(ctxsum_ref[...], axis=0, keepdims=True)
    hmean = (hxsum_ref[...] + 2.0 * msum) * inv_n
    ctx = ctxsum * inv_ne
    h = jnp.maximum(
        jnp.dot(hmean, wr1at_ref[...], preferred_element_type=jnp.float32)
        + jnp.dot(ctx, wr1bt_ref[...], preferred_element_type=jnp.float32)
        + br1_ref[...], 0.0)
    out_ref[...] = (jnp.dot(h, wr2t_ref[...],
                            preferred_element_type=jnp.float32) + br2_ref[...])


def _stage4(hxsum, msum, ctxsum, Wr1at, Wr1bt, br1, Wr2t, br2, n, ne):
    body = functools.partial(_stage4_body, inv_n=1.0 / n,
                             inv_ne=1.0 / (ne + 1e-06))
    return pl.pallas_call(
        body,
        out_shape=jax.ShapeDtypeStruct((1, 2), jnp.float32),
    )(hxsum, msum, ctxsum, Wr1at, Wr1bt, br1, Wr2t, br2)


def kernel(X, edges, E, W1, b1, W2, b2, W3, b3, W4, b4, Wp, bp,
           Wr1, br1, Wr2, br2, gate_scale):
    n = X.shape[0]
    ne = edges.shape[0]

    W1t = W1.T
    W2t = W2.T
    W3at = W3[:, :ND].T
    W3bt = W3[:, ND:2 * ND].T
    W3ct = W3[:, 2 * ND:].T
    W4t = W4.T
    Wpt = Wp.T
    Wr1at = Wr1[:, :HID].T
    Wr1bt = Wr1[:, HID:].T
    Wr2t = Wr2.T
    b1r = b1[None, :]
    b2r = b2[None, :]
    b3r = b3[None, :]
    b4r = b4[None, :]
    bpr = bp[None, :]
    br1r = br1[None, :]
    br2r = br2[None, :]
    gs = jnp.reshape(gate_scale.astype(jnp.float32), (1, 1))

    unit = 32 * K * NBUF
    nunits = (ne + unit - 1) // unit
    assert nunits == sum(PHASES), (nunits, PHASES)
    nep = nunits * unit
    src = edges[:, 0]
    dst = edges[:, 1]
    pad = (jnp.arange(nep - ne, dtype=jnp.int32) * 8) % n
    srcp = jnp.concatenate([src, pad])
    dstp = jnp.concatenate([dst, pad])
    Et = jnp.concatenate([E.T, jnp.zeros((ED, nep - ne), jnp.float32)], axis=1)
    mask = (jnp.arange(nep, dtype=jnp.int32) < ne).astype(jnp.float32)[None, :]

    A, B, hxsum = _stage1(X, W1t, b1r, W2t, b2r, W3at, W3bt)

    msums, ctxs = [], []
    lo = 0
    for u in PHASES:
        sl = slice(lo, lo + u * unit)
        lo += u * unit
        G = _stage2(A, B, srcp[sl], dstp[sl])
        ms, cs = _stage3(G, Et[:, sl], mask[:, sl], W3ct, b3r, W4t, b4r,
                         Wpt, bpr, gs)
        msums.append(ms)
        ctxs.append(cs)
    msum = jnp.concatenate(msums, axis=0)
    ctxsum = jnp.concatenate(ctxs, axis=0)
    return _stage4(hxsum, msum, ctxsum, Wr1at, Wr1bt, br1r, Wr2t, br2r, n, ne)


# speedup vs baseline: 6.2171x; 1.0182x over previous
import functools

import jax
import jax.numpy as jnp
from jax import lax
from jax.experimental import pallas as pl
from jax.experimental.pallas import tpu as pltpu
from jax.experimental.pallas import tpu_sc as plsc

ND = 128
HID = 128
ED = 4
NB = 2000
EB = 4096
K = 64
PHASES = (3, 4, 5, 8)

_DN_RHS_T = (((1,), (1,)), ((), ()))


def _stage1_body(x_ref, w1_ref, b1_ref, w2_ref, b2_ref, w3a_ref, w3b_ref,
                 a_ref, b_ref, hxsum_ref):
    i = pl.program_id(0)
    x = x_ref[...]
    h = jnp.maximum(
        lax.dot_general(x, w1_ref[...], _DN_RHS_T,
                        preferred_element_type=jnp.float32) + b1_ref[...], 0.0)
    hx = jnp.maximum(
        lax.dot_general(h, w2_ref[...], _DN_RHS_T,
                        preferred_element_type=jnp.float32) + b2_ref[...], 0.0)
    a_ref[...] = lax.dot_general(x, w3a_ref[...], _DN_RHS_T,
                                 preferred_element_type=jnp.float32)
    b_ref[...] = lax.dot_general(x, w3b_ref[...], _DN_RHS_T,
                                 preferred_element_type=jnp.float32)

    @pl.when(i == 0)
    def _():
        hxsum_ref[...] = jnp.zeros_like(hxsum_ref)

    hxsum_ref[...] += jnp.sum(hx, axis=0, keepdims=True)


def _stage1(X, W1t, b1, W2t, b2, W3at, W3bt):
    n = X.shape[0]
    grid = n // NB
    full = lambda i: (0, 0)
    return pl.pallas_call(
        _stage1_body,
        grid=(grid,),
        in_specs=[
            pl.BlockSpec((NB, ND), lambda i: (i, 0)),
            pl.BlockSpec((ND, HID), full),
            pl.BlockSpec((1, HID), full),
            pl.BlockSpec((HID, HID), full),
            pl.BlockSpec((1, HID), full),
            pl.BlockSpec((ND, HID), full),
            pl.BlockSpec((ND, HID), full),
        ],
        out_specs=[
            pl.BlockSpec((NB, HID), lambda i: (i, 0)),
            pl.BlockSpec((NB, HID), lambda i: (i, 0)),
            pl.BlockSpec((1, HID), full),
        ],
        out_shape=[
            jax.ShapeDtypeStruct((n, HID), jnp.float32),
            jax.ShapeDtypeStruct((n, HID), jnp.float32),
            jax.ShapeDtypeStruct((1, HID), jnp.float32),
        ],
    )(X, W1t, b1, W2t, b2, W3at, W3bt)


NBUF = 4
PW = HID


def _stage2(A, B, srcp, dstp):
    info = plsc.get_sparse_core_info()
    nc, ns = info.num_cores, info.num_subcores
    nw = nc * ns
    nep = srcp.shape[0]
    epw = nep // nw
    nch = epw // K

    mesh = plsc.VectorSubcoreMesh(core_axis_name="c", subcore_axis_name="s")

    @functools.partial(
        pl.kernel,
        mesh=mesh,
        out_type=jax.ShapeDtypeStruct((nep, PW), jnp.float32),
        scratch_types=(
            [pltpu.VMEM((epw,), jnp.int32)] * 2
            + [pltpu.VMEM((K, PW), jnp.float32)] * (2 * NBUF)
            + [pltpu.SemaphoreType.DMA] * (3 * NBUF)
        ),
    )
    def gather_kernel(a_hbm, b_hbm, src_hbm, dst_hbm, g_hbm,
                      si_v, di_v, *bufs_and_sems):
        ra = bufs_and_sems[0:NBUF]
        rb = bufs_and_sems[NBUF:2 * NBUF]
        sa = bufs_and_sems[2 * NBUF:3 * NBUF]
        sb = bufs_and_sems[3 * NBUF:4 * NBUF]
        wa = bufs_and_sems[4 * NBUF:5 * NBUF]

        wid = lax.axis_index("s") * nc + lax.axis_index("c")
        base = pl.multiple_of(wid * epw, K * 8)

        pltpu.sync_copy(src_hbm.at[pl.ds(base, epw)], si_v)
        pltpu.sync_copy(dst_hbm.at[pl.ds(base, epw)], di_v)

        def _idx(slab, j):
            return slab.at[pl.ds(pl.multiple_of(j * K, 8), K)]

        def start_gather(j, b):
            pltpu.make_async_copy(a_hbm.at[_idx(si_v, j)], ra[b], sa[b]).start()
            pltpu.make_async_copy(b_hbm.at[_idx(di_v, j)], rb[b], sb[b]).start()

        def wait_gather(j, b):
            pltpu.make_async_copy(a_hbm.at[_idx(si_v, j)], ra[b], sa[b]).wait()
            pltpu.make_async_copy(b_hbm.at[_idx(di_v, j)], rb[b], sb[b]).wait()

        def add_bufs(b):
            @pl.loop(0, K)
            def _(r):
                for c in range(PW // 16):
                    sl = pl.ds(c * 16, 16)
                    ra[b][r, sl] = ra[b][r, sl] + rb[b][r, sl]

        def start_wb(j, b):
            off = pl.multiple_of(base + j * K, 8)
            pltpu.make_async_copy(ra[b], g_hbm.at[pl.ds(off, K)], wa[b]).start()

        def wait_wb(j, b):
            off = pl.multiple_of(base + j * K, 8)
            pltpu.make_async_copy(ra[b], g_hbm.at[pl.ds(off, K)], wa[b]).wait()

        for p in range(NBUF - 1):
            start_gather(p, p)

        @pl.loop(0, nch, step=NBUF)
        def _(j0):
            for b in range(NBUF):
                j = j0 + b
                wait_gather(j, b)
                add_bufs(b)
                start_wb(j, b)
                nb = (b + NBUF - 1) % NBUF

                @pl.when(j > 0)
                def _():
                    wait_wb(j - 1, nb)

                q = j + NBUF - 1

                @pl.when(q < nch)
                def _():
                    start_gather(q, nb)

        wait_wb(nch - 1, (nch - 1) % NBUF)

    return gather_kernel(A, B, srcp, dstp)


def _stage3_body(g_ref, et_ref, mask_ref, w3c_ref, b3_ref, w4_ref,
                 b4_ref, wp_ref, bp_ref, gs_ref, msum_ref, ctx_ref):
    i = pl.program_id(0)
    et = et_ref[...]
    gate = jnp.clip(1.0 + gs_ref[0, 0] * et[2:3, :], 0.0, 3.0) * mask_ref[...]
    dn = (((0,), (1,)), ((), ()))
    ec = lax.dot_general(et, w3c_ref[...], dn,
                         preferred_element_type=jnp.float32)
    h1 = jnp.maximum(g_ref[...] + ec + b3_ref[...], 0.0)
    mm = jnp.maximum(
        lax.dot_general(h1, w4_ref[...], _DN_RHS_T,
                        preferred_element_type=jnp.float32) + b4_ref[...], 0.0)
    p = lax.dot_general(et, wp_ref[...], dn,
                        preferred_element_type=jnp.float32) + bp_ref[...]

    @pl.when(i == 0)
    def _():
        msum_ref[...] = jnp.zeros_like(msum_ref)
        ctx_ref[...] = jnp.zeros_like(ctx_ref)

    msum_ref[...] += jnp.dot(gate, mm, preferred_element_type=jnp.float32)
    ctx_ref[...] += jnp.dot(gate, p, preferred_element_type=jnp.float32)


def _stage3(G, Et, mask, W3c, b3, W4, b4, Wp, bp, gs):
    npe = G.shape[0]
    grid = npe // EB
    full = lambda i: (0, 0)
    return pl.pallas_call(
        _stage3_body,
        grid=(grid,),
        in_specs=[
            pl.BlockSpec((EB, HID), lambda i: (i, 0)),
            pl.BlockSpec((4, EB), lambda i: (0, i)),
            pl.BlockSpec((1, EB), lambda i: (0, i)),
            pl.BlockSpec((HID, 4), full),
            pl.BlockSpec((1, HID), full),
            pl.BlockSpec((HID, HID), full),
            pl.BlockSpec((1, HID), full),
            pl.BlockSpec((HID, 4), full),
            pl.BlockSpec((1, HID), full),
            pl.BlockSpec((1, 1), full),
        ],
        out_specs=[
            pl.BlockSpec((1, HID), full),
            pl.BlockSpec((1, HID), full),
        ],
        out_shape=[
            jax.ShapeDtypeStruct((1, HID), jnp.float32),
            jax.ShapeDtypeStruct((1, HID), jnp.float32),
        ],
    )(G, Et, mask, W3c, b3, W4, b4, Wp, bp, gs)


def _stage4_body(hxsum_ref, msum_ref, ctxsum_ref, wr1a_ref, wr1b_ref,
                 br1_ref, wr2_ref, br2_ref, out_ref, *, inv_n, inv_ne):
    msum = jnp.sum(msum_ref[...], axis=0, keepdims=True)
    ctxsum = jnp.sum(ctxsum_ref[...], axis=0, keepdims=True)
    hmean = (hxsum_ref[...] + 2.0 * msum) * inv_n
    ctx = ctxsum * inv_ne
    h = jnp.maximum(
        lax.dot_general(hmean, wr1a_ref[...], _DN_RHS_T,
                        preferred_element_type=jnp.float32)
        + lax.dot_general(ctx, wr1b_ref[...], _DN_RHS_T,
                          preferred_element_type=jnp.float32)
        + br1_ref[...], 0.0)
    out_ref[...] = (lax.dot_general(h, wr2_ref[...], _DN_RHS_T,
                                    preferred_element_type=jnp.float32)
                    + br2_ref[...])


def _stage4(hxsum, msum, ctxsum, Wr1a, Wr1b, br1, Wr2, br2, n, ne):
    body = functools.partial(_stage4_body, inv_n=1.0 / n,
                             inv_ne=1.0 / (ne + 1e-06))
    return pl.pallas_call(
        body,
        out_shape=jax.ShapeDtypeStruct((1, 2), jnp.float32),
    )(hxsum, msum, ctxsum, Wr1a, Wr1b, br1, Wr2, br2)


def kernel(X, edges, E, W1, b1, W2, b2, W3, b3, W4, b4, Wp, bp,
           Wr1, br1, Wr2, br2, gate_scale):
    n = X.shape[0]
    ne = edges.shape[0]

    W3a = W3[:, :ND]
    W3b = W3[:, ND:2 * ND]
    W3c = W3[:, 2 * ND:]
    Wr1a = Wr1[:, :HID]
    Wr1b = Wr1[:, HID:]
    b1r = b1[None, :]
    b2r = b2[None, :]
    b3r = b3[None, :]
    b4r = b4[None, :]
    bpr = bp[None, :]
    br1r = br1[None, :]
    br2r = br2[None, :]
    gs = jnp.reshape(gate_scale.astype(jnp.float32), (1, 1))

    unit = 32 * K * NBUF
    nunits = (ne + unit - 1) // unit
    assert nunits == sum(PHASES), (nunits, PHASES)
    nep = nunits * unit
    src = edges[:, 0]
    dst = edges[:, 1]
    pad = (jnp.arange(nep - ne, dtype=jnp.int32) * 8) % n
    srcp = jnp.concatenate([src, pad])
    dstp = jnp.concatenate([dst, pad])
    Et = jnp.concatenate([E.T, jnp.zeros((ED, nep - ne), jnp.float32)], axis=1)
    mask = (jnp.arange(nep, dtype=jnp.int32) < ne).astype(jnp.float32)[None, :]

    A, B, hxsum = _stage1(X, W1, b1r, W2, b2r, W3a, W3b)

    msums, ctxs = [], []
    lo = 0
    for u in PHASES:
        sz = u * unit
        sl = slice(lo, lo + sz)
        lo += sz
        G = _stage2(A, B, srcp[sl], dstp[sl])
        ms, cs = _stage3(G, Et[:, sl], mask[:, sl], W3c, b3r, W4, b4r,
                         Wp, bpr, gs)
        msums.append(ms)
        ctxs.append(cs)
    msum = jnp.concatenate(msums, axis=0)
    ctxsum = jnp.concatenate(ctxs, axis=0)
    return _stage4(hxsum, msum, ctxsum, Wr1a, Wr1b, br1r, Wr2, br2r, n, ne)
